# Initial kernel scaffold; baseline (speedup 1.0000x reference)
import jax
import jax.numpy as jnp
from jax.experimental import pallas as pl


def kernel(x, edge_index, W1, a_src1, a_dst1, b1, W2, a_src2, a_dst2, b2):
    raise NotImplementedError("write your pallas kernel here")


# speedup vs baseline: 19.9050x; 19.9050x over previous
import functools

import jax
import jax.numpy as jnp
from jax import lax
from jax.experimental import pallas as pl
from jax.experimental.pallas import tpu as pltpu
from jax.experimental.pallas import tpu_sc as plsc

D = 128
NPAD = 10240
NC, NS, LANES = 2, 16, 16
NW = NC * NS
CHUNK = 128
ROWS_PER_TILE = NPAD // NS
NEG_SLOPE = 0.2
ROWBLK = 2000


def _leaky(t):
    return jnp.maximum(t, NEG_SLOPE * t)


def _pre_body(x_ref, w_ref, as_ref, ad_ref, h_ref, asrc_ref, adst_ref,
              gmax_ref):
    h = jnp.dot(x_ref[...], w_ref[...], preferred_element_type=jnp.float32)
    h_ref[...] = h
    asv = jnp.sum(h * as_ref[...], axis=1, keepdims=True)
    adv = jnp.sum(h * ad_ref[...], axis=1, keepdims=True)
    asrc_ref[...] = asv
    adst_ref[...] = adv

    @pl.when(pl.program_id(0) == 0)
    def _():
        gmax_ref[...] = jnp.full((1, 1), -jnp.inf, jnp.float32)

    gmax_ref[...] = jnp.maximum(gmax_ref[...], jnp.max(asv))


def _gat_pre(x, W, a_s, a_d):
    n = x.shape[0]
    grid = n // ROWBLK
    return pl.pallas_call(
        _pre_body,
        grid=(grid,),
        in_specs=[
            pl.BlockSpec((ROWBLK, D), lambda i: (i, 0)),
            pl.BlockSpec((D, D), lambda i: (0, 0)),
            pl.BlockSpec((1, D), lambda i: (0, 0)),
            pl.BlockSpec((1, D), lambda i: (0, 0)),
        ],
        out_specs=[
            pl.BlockSpec((ROWBLK, D), lambda i: (i, 0)),
            pl.BlockSpec((ROWBLK, 1), lambda i: (i, 0)),
            pl.BlockSpec((ROWBLK, 1), lambda i: (i, 0)),
            pl.BlockSpec((1, 1), lambda i: (0, 0)),
        ],
        out_shape=[
            jax.ShapeDtypeStruct((n, D), jnp.float32),
            jax.ShapeDtypeStruct((n, 1), jnp.float32),
            jax.ShapeDtypeStruct((n, 1), jnp.float32),
            jax.ShapeDtypeStruct((1, 1), jnp.float32),
        ],
    )(x, W, a_s, a_d)


def _combine(acc_a, acc_b, den_a, den_b, h, asrc, adst, gmax):
    t = asrc[...] + adst[...]
    t2 = gmax[...] + adst[...]
    wself = jnp.exp(_leaky(t) - _leaky(t2))
    den = den_a[0] + den_b[0] + wself + 1e-16
    num = acc_a[0] + acc_b[0] + wself * h[...]
    return num / den


def _mid_body(acc_a, acc_b, den_a, den_b, h_ref, asrc, adst, gmax, b_ref,
              w_ref, as_ref, ad_ref, h2_ref, asrc2_ref, adst2_ref, gmax2_ref):
    out1 = jnp.maximum(
        _combine(acc_a, acc_b, den_a, den_b, h_ref, asrc, adst, gmax)
        + b_ref[...], 0.0)
    h2 = jnp.dot(out1, w_ref[...], preferred_element_type=jnp.float32)
    h2_ref[...] = h2
    asv = jnp.sum(h2 * as_ref[...], axis=1, keepdims=True)
    adv = jnp.sum(h2 * ad_ref[...], axis=1, keepdims=True)
    asrc2_ref[...] = asv
    adst2_ref[...] = adv

    @pl.when(pl.program_id(0) == 0)
    def _():
        gmax2_ref[...] = jnp.full((1, 1), -jnp.inf, jnp.float32)

    gmax2_ref[...] = jnp.maximum(gmax2_ref[...], jnp.max(asv))


def _gat_mid(acc, den, h, asrc, adst, gmax, b, W2, a_s2, a_d2):
    n = h.shape[0]
    grid = n // ROWBLK
    part = lambda c: pl.BlockSpec((1, ROWBLK, D), lambda i, c=c: (c, i, 0))
    partd = lambda c: pl.BlockSpec((1, ROWBLK, 1), lambda i, c=c: (c, i, 0))
    colv = pl.BlockSpec((ROWBLK, 1), lambda i: (i, 0))
    return pl.pallas_call(
        _mid_body,
        grid=(grid,),
        in_specs=[
            part(0), part(1), partd(0), partd(1),
            pl.BlockSpec((ROWBLK, D), lambda i: (i, 0)),
            colv, colv,
            pl.BlockSpec((1, 1), lambda i: (0, 0)),
            pl.BlockSpec((1, D), lambda i: (0, 0)),
            pl.BlockSpec((D, D), lambda i: (0, 0)),
            pl.BlockSpec((1, D), lambda i: (0, 0)),
            pl.BlockSpec((1, D), lambda i: (0, 0)),
        ],
        out_specs=[
            pl.BlockSpec((ROWBLK, D), lambda i: (i, 0)),
            colv, colv,
            pl.BlockSpec((1, 1), lambda i: (0, 0)),
        ],
        out_shape=[
            jax.ShapeDtypeStruct((n, D), jnp.float32),
            jax.ShapeDtypeStruct((n, 1), jnp.float32),
            jax.ShapeDtypeStruct((n, 1), jnp.float32),
            jax.ShapeDtypeStruct((1, 1), jnp.float32),
        ],
    )(acc, acc, den, den, h, asrc, adst, gmax, b, W2, a_s2, a_d2)


def _fin_body(acc_a, acc_b, den_a, den_b, h_ref, asrc, adst, gmax, b_ref,
              out_ref):
    out_ref[...] = (_combine(acc_a, acc_b, den_a, den_b, h_ref, asrc, adst,
                             gmax) + b_ref[...])


def _gat_fin(acc, den, h, asrc, adst, gmax, b):
    n = h.shape[0]
    grid = n // ROWBLK
    part = lambda c: pl.BlockSpec((1, ROWBLK, D), lambda i, c=c: (c, i, 0))
    partd = lambda c: pl.BlockSpec((1, ROWBLK, 1), lambda i, c=c: (c, i, 0))
    colv = pl.BlockSpec((ROWBLK, 1), lambda i: (i, 0))
    return pl.pallas_call(
        _fin_body,
        grid=(grid,),
        in_specs=[
            part(0), part(1), partd(0), partd(1),
            pl.BlockSpec((ROWBLK, D), lambda i: (i, 0)),
            colv, colv,
            pl.BlockSpec((1, 1), lambda i: (0, 0)),
            pl.BlockSpec((1, D), lambda i: (0, 0)),
        ],
        out_specs=pl.BlockSpec((ROWBLK, D), lambda i: (i, 0)),
        out_shape=jax.ShapeDtypeStruct((n, D), jnp.float32),
    )(acc, acc, den, den, h, asrc, adst, gmax, b)


def _make_edge_kernel(num_chunks):
    mesh = plsc.VectorSubcoreMesh(core_axis_name="c", subcore_axis_name="s",
                                  num_cores=NC, num_subcores=NS)

    @functools.partial(
        pl.kernel,
        out_type=(jax.ShapeDtypeStruct((NC, NPAD, D), jnp.float32),
                  jax.ShapeDtypeStruct((NC, NPAD), jnp.float32)),
        mesh=mesh,
        compiler_params=pltpu.CompilerParams(needs_layout_passes=False),
        scratch_types=[
            pltpu.VMEM_SHARED((NPAD, D), jnp.float32),
            pltpu.VMEM_SHARED((NPAD,), jnp.float32),
            pltpu.VMEM((NPAD,), jnp.float32),
            pltpu.VMEM((NPAD,), jnp.float32),
            pltpu.VMEM((LANES,), jnp.float32),
            pltpu.VMEM((CHUNK,), jnp.int32),
            pltpu.VMEM((CHUNK,), jnp.int32),
            pltpu.VMEM((CHUNK,), jnp.float32),
            pltpu.VMEM((CHUNK, D), jnp.float32),
            pltpu.VMEM((ROWS_PER_TILE,), jnp.float32),
            pltpu.SemaphoreType.DMA,
        ],
    )
    def edge_kernel(h_hbm, asrc_hbm, adst_hbm, gmax_hbm, src_hbm, dst_hbm,
                    acc_o, den_o, accS, denS, asrc_t, adst_t, gmax_t, sidx,
                    didx, wv, rows, zb, sem):
        c = lax.axis_index("c")
        s = lax.axis_index("s")
        gw = c * NS + s
        pltpu.sync_copy(asrc_hbm, asrc_t)
        pltpu.sync_copy(adst_hbm, adst_t)
        pltpu.sync_copy(gmax_hbm, gmax_t)

        zero16 = jnp.zeros((LANES,), jnp.float32)

        def _zrow(r, _):
            for j in range(D // LANES):
                rows[r, pl.ds(j * LANES, LANES)] = zero16
            return 0

        lax.fori_loop(0, CHUNK, _zrow, 0)

        def _zzb(i, _):
            zb[pl.ds(i * LANES, LANES)] = zero16
            return 0

        lax.fori_loop(0, ROWS_PER_TILE // LANES, _zzb, 0)

        row0 = s * ROWS_PER_TILE
        for j in range(ROWS_PER_TILE // CHUNK):
            pltpu.sync_copy(rows, accS.at[pl.ds(row0 + j * CHUNK, CHUNK)])
        pltpu.sync_copy(zb, denS.at[pl.ds(row0, ROWS_PER_TILE)])
        plsc.subcore_barrier()

        gv = gmax_t[...]
        ebase = gw * (num_chunks * CHUNK)

        def _chunk(ci, _):
            base = ebase + ci * CHUNK
            pltpu.sync_copy(src_hbm.at[pl.ds(base, CHUNK)], sidx)
            pltpu.sync_copy(dst_hbm.at[pl.ds(base, CHUNK)], didx)
            pltpu.async_copy(h_hbm.at[sidx], rows, sem).wait()

            def _wgrp(j, _):
                off = pl.multiple_of(j * LANES, LANES)
                si = sidx[pl.ds(off, LANES)]
                di = didx[pl.ds(off, LANES)]
                av = plsc.load_gather(asrc_t, [si])
                dv = plsc.load_gather(adst_t, [di])
                wv[pl.ds(off, LANES)] = jnp.exp(
                    _leaky(av + dv) - _leaky(gv + dv))
                return 0

            lax.fori_loop(0, CHUNK // LANES, _wgrp, 0)

            def _scale(g, _):
                off = pl.multiple_of(g * LANES, LANES)
                wg = wv[pl.ds(off, LANES)]
                for k in range(LANES):
                    wk = wg[k]
                    for j in range(D // LANES):
                        sl = pl.ds(j * LANES, LANES)
                        rows[off + k, sl] = rows[off + k, sl] * wk
                return 0

            lax.fori_loop(0, CHUNK // LANES, _scale, 0)

            pltpu.sync_copy(wv, denS.at[didx], add=True)
            pltpu.sync_copy(rows, accS.at[didx], add=True)
            return 0

        lax.fori_loop(0, num_chunks, _chunk, 0)
        plsc.subcore_barrier()
        pltpu.sync_copy(accS.at[pl.ds(row0, ROWS_PER_TILE)],
                        acc_o.at[c, pl.ds(row0, ROWS_PER_TILE)])
        pltpu.sync_copy(denS.at[pl.ds(row0, ROWS_PER_TILE)],
                        den_o.at[c, pl.ds(row0, ROWS_PER_TILE)])

    return edge_kernel


def kernel(x, edge_index, W1, a_src1, a_dst1, b1, W2, a_src2, a_dst2, b2):
    n = x.shape[0]
    e = edge_index.shape[1]
    src = edge_index[0]
    dst = edge_index[1]
    num_chunks = -(-e // (NW * CHUNK))
    epad = NW * num_chunks * CHUNK
    src_p = jnp.concatenate([src, jnp.zeros((epad - e,), jnp.int32)])
    dst_p = jnp.concatenate([dst, jnp.full((epad - e,), n, jnp.int32)])
    pad_n = NPAD - n

    a_s1 = a_src1.reshape(1, D)
    a_d1 = a_dst1.reshape(1, D)
    a_s2 = a_src2.reshape(1, D)
    a_d2 = a_dst2.reshape(1, D)
    b1r = b1.reshape(1, D)
    b2r = b2.reshape(1, D)

    edge_k = _make_edge_kernel(num_chunks)

    def pad_col(v):
        return jnp.concatenate([v[:, 0], jnp.zeros((pad_n,), jnp.float32)])

    h1, asrc1, adst1, gmax1 = _gat_pre(x, W1, a_s1, a_d1)
    g16 = jnp.broadcast_to(gmax1.reshape(()), (LANES,))
    acc1, den1 = edge_k(h1, pad_col(asrc1), pad_col(adst1), g16, src_p, dst_p)
    h2, asrc2, adst2, gmax2 = _gat_mid(acc1, den1.reshape(NC, NPAD, 1), h1,
                                       asrc1, adst1, gmax1, b1r, W2, a_s2,
                                       a_d2)
    g16b = jnp.broadcast_to(gmax2.reshape(()), (LANES,))
    acc2, den2 = edge_k(h2, pad_col(asrc2), pad_col(adst2), g16b, src_p,
                        dst_p)
    return _gat_fin(acc2, den2.reshape(NC, NPAD, 1), h2, asrc2, adst2, gmax2,
                    b2r)


# speedup vs baseline: 20.2591x; 1.0178x over previous
import functools

import jax
import jax.numpy as jnp
from jax import lax
from jax.experimental import pallas as pl
from jax.experimental.pallas import tpu as pltpu
from jax.experimental.pallas import tpu_sc as plsc

D = 128
NPAD = 10240
NC, NS, LANES = 2, 16, 16
NW = NC * NS
CHUNK = 128
ROWS_PER_TILE = NPAD // NS
NEG_SLOPE = 0.2
ROWBLK = 2000
DEPTH = 2


def _leaky(t):
    return jnp.maximum(t, NEG_SLOPE * t)


def _pre_body(x_ref, w_ref, as_ref, ad_ref, h_ref, asrc_ref, adst_ref,
              gmax_ref):
    h = jnp.dot(x_ref[...], w_ref[...], preferred_element_type=jnp.float32)
    h_ref[...] = h
    asv = jnp.sum(h * as_ref[...], axis=1, keepdims=True)
    adv = jnp.sum(h * ad_ref[...], axis=1, keepdims=True)
    asrc_ref[...] = asv
    adst_ref[...] = adv

    @pl.when(pl.program_id(0) == 0)
    def _():
        gmax_ref[...] = jnp.full((1, 1), -jnp.inf, jnp.float32)

    gmax_ref[...] = jnp.maximum(gmax_ref[...], jnp.max(asv))


def _gat_pre(x, W, a_s, a_d):
    n = x.shape[0]
    grid = n // ROWBLK
    return pl.pallas_call(
        _pre_body,
        grid=(grid,),
        in_specs=[
            pl.BlockSpec((ROWBLK, D), lambda i: (i, 0)),
            pl.BlockSpec((D, D), lambda i: (0, 0)),
            pl.BlockSpec((1, D), lambda i: (0, 0)),
            pl.BlockSpec((1, D), lambda i: (0, 0)),
        ],
        out_specs=[
            pl.BlockSpec((ROWBLK, D), lambda i: (i, 0)),
            pl.BlockSpec((ROWBLK, 1), lambda i: (i, 0)),
            pl.BlockSpec((ROWBLK, 1), lambda i: (i, 0)),
            pl.BlockSpec((1, 1), lambda i: (0, 0)),
        ],
        out_shape=[
            jax.ShapeDtypeStruct((n, D), jnp.float32),
            jax.ShapeDtypeStruct((n, 1), jnp.float32),
            jax.ShapeDtypeStruct((n, 1), jnp.float32),
            jax.ShapeDtypeStruct((1, 1), jnp.float32),
        ],
    )(x, W, a_s, a_d)


def _combine(acc_a, acc_b, den_a, den_b, h, asrc, adst, gmax):
    t = asrc[...] + adst[...]
    t2 = gmax[...] + adst[...]
    wself = jnp.exp(_leaky(t) - _leaky(t2))
    den = den_a[0] + den_b[0] + wself + 1e-16
    num = acc_a[0] + acc_b[0] + wself * h[...]
    return num / den


def _mid_body(acc_a, acc_b, den_a, den_b, h_ref, asrc, adst, gmax, b_ref,
              w_ref, as_ref, ad_ref, h2_ref, asrc2_ref, adst2_ref, gmax2_ref):
    out1 = jnp.maximum(
        _combine(acc_a, acc_b, den_a, den_b, h_ref, asrc, adst, gmax)
        + b_ref[...], 0.0)
    h2 = jnp.dot(out1, w_ref[...], preferred_element_type=jnp.float32)
    h2_ref[...] = h2
    asv = jnp.sum(h2 * as_ref[...], axis=1, keepdims=True)
    adv = jnp.sum(h2 * ad_ref[...], axis=1, keepdims=True)
    asrc2_ref[...] = asv
    adst2_ref[...] = adv

    @pl.when(pl.program_id(0) == 0)
    def _():
        gmax2_ref[...] = jnp.full((1, 1), -jnp.inf, jnp.float32)

    gmax2_ref[...] = jnp.maximum(gmax2_ref[...], jnp.max(asv))


def _gat_mid(acc, den, h, asrc, adst, gmax, b, W2, a_s2, a_d2):
    n = h.shape[0]
    grid = n // ROWBLK
    part = lambda c: pl.BlockSpec((1, ROWBLK, D), lambda i, c=c: (c, i, 0))
    partd = lambda c: pl.BlockSpec((1, ROWBLK, 1), lambda i, c=c: (c, i, 0))
    colv = pl.BlockSpec((ROWBLK, 1), lambda i: (i, 0))
    return pl.pallas_call(
        _mid_body,
        grid=(grid,),
        in_specs=[
            part(0), part(1), partd(0), partd(1),
            pl.BlockSpec((ROWBLK, D), lambda i: (i, 0)),
            colv, colv,
            pl.BlockSpec((1, 1), lambda i: (0, 0)),
            pl.BlockSpec((1, D), lambda i: (0, 0)),
            pl.BlockSpec((D, D), lambda i: (0, 0)),
            pl.BlockSpec((1, D), lambda i: (0, 0)),
            pl.BlockSpec((1, D), lambda i: (0, 0)),
        ],
        out_specs=[
            pl.BlockSpec((ROWBLK, D), lambda i: (i, 0)),
            colv, colv,
            pl.BlockSpec((1, 1), lambda i: (0, 0)),
        ],
        out_shape=[
            jax.ShapeDtypeStruct((n, D), jnp.float32),
            jax.ShapeDtypeStruct((n, 1), jnp.float32),
            jax.ShapeDtypeStruct((n, 1), jnp.float32),
            jax.ShapeDtypeStruct((1, 1), jnp.float32),
        ],
    )(acc, acc, den, den, h, asrc, adst, gmax, b, W2, a_s2, a_d2)


def _fin_body(acc_a, acc_b, den_a, den_b, h_ref, asrc, adst, gmax, b_ref,
              out_ref):
    out_ref[...] = (_combine(acc_a, acc_b, den_a, den_b, h_ref, asrc, adst,
                             gmax) + b_ref[...])


def _gat_fin(acc, den, h, asrc, adst, gmax, b):
    n = h.shape[0]
    grid = n // ROWBLK
    part = lambda c: pl.BlockSpec((1, ROWBLK, D), lambda i, c=c: (c, i, 0))
    partd = lambda c: pl.BlockSpec((1, ROWBLK, 1), lambda i, c=c: (c, i, 0))
    colv = pl.BlockSpec((ROWBLK, 1), lambda i: (i, 0))
    return pl.pallas_call(
        _fin_body,
        grid=(grid,),
        in_specs=[
            part(0), part(1), partd(0), partd(1),
            pl.BlockSpec((ROWBLK, D), lambda i: (i, 0)),
            colv, colv,
            pl.BlockSpec((1, 1), lambda i: (0, 0)),
            pl.BlockSpec((1, D), lambda i: (0, 0)),
        ],
        out_specs=pl.BlockSpec((ROWBLK, D), lambda i: (i, 0)),
        out_shape=jax.ShapeDtypeStruct((n, D), jnp.float32),
    )(acc, acc, den, den, h, asrc, adst, gmax, b)


def _make_edge_kernel(num_chunks):
    assert num_chunks % DEPTH == 0
    mesh = plsc.VectorSubcoreMesh(core_axis_name="c", subcore_axis_name="s",
                                  num_cores=NC, num_subcores=NS)

    scratch = [
        pltpu.VMEM_SHARED((NPAD, D), jnp.float32),
        pltpu.VMEM_SHARED((NPAD,), jnp.float32),
        pltpu.VMEM_SHARED((NPAD,), jnp.float32),
        pltpu.VMEM_SHARED((NPAD,), jnp.float32),
        pltpu.VMEM((LANES,), jnp.float32),
        pltpu.VMEM((ROWS_PER_TILE,), jnp.float32),
    ]
    NBUF = 10
    for _ in range(DEPTH):
        scratch += [
            pltpu.VMEM((CHUNK,), jnp.int32),
            pltpu.VMEM((CHUNK,), jnp.int32),
            pltpu.VMEM((CHUNK,), jnp.float32),
            pltpu.VMEM((CHUNK, D), jnp.float32),
            pltpu.VMEM((CHUNK,), jnp.float32),
            pltpu.VMEM((CHUNK,), jnp.float32),
            pltpu.SemaphoreType.DMA,
            pltpu.SemaphoreType.DMA,
            pltpu.SemaphoreType.DMA,
            pltpu.SemaphoreType.DMA,
        ]

    @functools.partial(
        pl.kernel,
        out_type=(jax.ShapeDtypeStruct((NC, NPAD, D), jnp.float32),
                  jax.ShapeDtypeStruct((NC, NPAD), jnp.float32)),
        mesh=mesh,
        compiler_params=pltpu.CompilerParams(needs_layout_passes=False),
        scratch_types=scratch,
    )
    def edge_kernel(h_hbm, asrc_hbm, adst_hbm, gmax_hbm, src_hbm, dst_hbm,
                    acc_o, den_o, accS, denS, asrcS, adstS, gmax_t, zb,
                    *ring):
        bufs = [ring[NBUF * k:NBUF * (k + 1)] for k in range(DEPTH)]
        c = lax.axis_index("c")
        s = lax.axis_index("s")
        gw = c * NS + s

        @pl.when(s == 0)
        def _():
            pltpu.sync_copy(asrc_hbm, asrcS)
            pltpu.sync_copy(adst_hbm, adstS)

        pltpu.sync_copy(gmax_hbm, gmax_t)

        zero16 = jnp.zeros((LANES,), jnp.float32)
        rows0 = bufs[0][3]

        def _zrow(r, _):
            for j in range(D // LANES):
                rows0[r, pl.ds(j * LANES, LANES)] = zero16
            return 0

        lax.fori_loop(0, CHUNK, _zrow, 0)

        def _zzb(i, _):
            zb[pl.ds(i * LANES, LANES)] = zero16
            return 0

        lax.fori_loop(0, ROWS_PER_TILE // LANES, _zzb, 0)

        row0 = s * ROWS_PER_TILE
        for j in range(ROWS_PER_TILE // CHUNK):
            pltpu.sync_copy(rows0, accS.at[pl.ds(row0 + j * CHUNK, CHUNK)])
        pltpu.sync_copy(zb, denS.at[pl.ds(row0, ROWS_PER_TILE)])
        plsc.subcore_barrier()

        gv = gmax_t[...]
        ebase = gw * (num_chunks * CHUNK)

        def _stage(ci, b, first):
            sidx, didx, wv, rows, av, adv, semg, sema, semd, semt = bufs[b]
            if not first:
                pltpu.make_async_copy(rows, accS.at[didx], sema).wait()
                pltpu.make_async_copy(wv, denS.at[didx], semd).wait()
            base = ebase + ci * CHUNK
            pltpu.sync_copy(src_hbm.at[pl.ds(base, CHUNK)], sidx)
            pltpu.sync_copy(dst_hbm.at[pl.ds(base, CHUNK)], didx)
            pltpu.async_copy(asrcS.at[sidx], av, semt)
            pltpu.async_copy(adstS.at[didx], adv, semt)
            pltpu.async_copy(h_hbm.at[sidx], rows, semg)

        def _proc(b):
            sidx, didx, wv, rows, av, adv, semg, sema, semd, semt = bufs[b]
            pltpu.make_async_copy(asrcS.at[sidx], av, semt).wait()
            pltpu.make_async_copy(adstS.at[didx], adv, semt).wait()

            def _wgrp(j, _):
                off = pl.multiple_of(j * LANES, LANES)
                avv = av[pl.ds(off, LANES)]
                advv = adv[pl.ds(off, LANES)]
                wv[pl.ds(off, LANES)] = jnp.exp(
                    _leaky(avv + advv) - _leaky(gv + advv))
                return 0

            lax.fori_loop(0, CHUNK // LANES, _wgrp, 0)
            pltpu.make_async_copy(h_hbm.at[sidx], rows, semg).wait()

            def _scale(g, _):
                off = pl.multiple_of(g * LANES, LANES)
                wg = wv[pl.ds(off, LANES)]
                for k in range(LANES):
                    wk = wg[k]
                    for j in range(D // LANES):
                        sl = pl.ds(j * LANES, LANES)
                        rows[off + k, sl] = rows[off + k, sl] * wk
                return 0

            lax.fori_loop(0, CHUNK // LANES, _scale, 0)

            pltpu.async_copy(rows, accS.at[didx], sema, add=True)
            pltpu.async_copy(wv, denS.at[didx], semd, add=True)

        for k in range(DEPTH):
            _stage(k, k, True)

        ngroups = num_chunks // DEPTH

        def _group(g, _):
            i0 = g * DEPTH
            not_last = g < ngroups - 1
            for k in range(DEPTH):
                _proc(k)

                @pl.when(not_last)
                def _(k=k):
                    _stage(i0 + DEPTH + k, k, False)

            return 0

        lax.fori_loop(0, ngroups, _group, 0)

        for k in range(DEPTH):
            sidx, didx, wv, rows, av, adv, semg, sema, semd, semt = bufs[k]
            pltpu.make_async_copy(rows, accS.at[didx], sema).wait()
            pltpu.make_async_copy(wv, denS.at[didx], semd).wait()

        plsc.subcore_barrier()
        pltpu.sync_copy(accS.at[pl.ds(row0, ROWS_PER_TILE)],
                        acc_o.at[c, pl.ds(row0, ROWS_PER_TILE)])
        pltpu.sync_copy(denS.at[pl.ds(row0, ROWS_PER_TILE)],
                        den_o.at[c, pl.ds(row0, ROWS_PER_TILE)])

    return edge_kernel


def kernel(x, edge_index, W1, a_src1, a_dst1, b1, W2, a_src2, a_dst2, b2):
    n = x.shape[0]
    e = edge_index.shape[1]
    src = edge_index[0]
    dst = edge_index[1]
    num_chunks = -(-e // (NW * CHUNK * DEPTH)) * DEPTH
    epad = NW * num_chunks * CHUNK
    src_p = jnp.concatenate([src, jnp.zeros((epad - e,), jnp.int32)])
    dst_p = jnp.concatenate([dst, jnp.full((epad - e,), n, jnp.int32)])
    pad_n = NPAD - n

    a_s1 = a_src1.reshape(1, D)
    a_d1 = a_dst1.reshape(1, D)
    a_s2 = a_src2.reshape(1, D)
    a_d2 = a_dst2.reshape(1, D)
    b1r = b1.reshape(1, D)
    b2r = b2.reshape(1, D)

    edge_k = _make_edge_kernel(num_chunks)

    def pad_col(v):
        return jnp.concatenate([v[:, 0], jnp.zeros((pad_n,), jnp.float32)])

    h1, asrc1, adst1, gmax1 = _gat_pre(x, W1, a_s1, a_d1)
    g16 = jnp.broadcast_to(gmax1.reshape(()), (LANES,))
    acc1, den1 = edge_k(h1, pad_col(asrc1), pad_col(adst1), g16, src_p, dst_p)
    h2, asrc2, adst2, gmax2 = _gat_mid(acc1, den1.reshape(NC, NPAD, 1), h1,
                                       asrc1, adst1, gmax1, b1r, W2, a_s2,
                                       a_d2)
    g16b = jnp.broadcast_to(gmax2.reshape(()), (LANES,))
    acc2, den2 = edge_k(h2, pad_col(asrc2), pad_col(adst2), g16b, src_p,
                        dst_p)
    return _gat_fin(acc2, den2.reshape(NC, NPAD, 1), h2, asrc2, adst2, gmax2,
                    b2r)
